# baseline (speedup 1.0000x reference)
import math

import jax
import jax.numpy as jnp
from jax import lax
from jax.experimental import pallas as pl
from jax.experimental.pallas import tpu as pltpu
from jax.experimental.pallas import tpu_sc as plsc

BOHR = 0.5291772108
N_ATOMS = 10000
N_EDGES = 640000
NZ = 95
NREF = 25
ROW = 80
NTILES = 16
NCORES = 2

EA = N_EDGES // NTILES
CA = 4000
EB = N_EDGES // (NCORES * NTILES)
CB = 400
NCH = EB // CB
ZCH = 2000

S6 = 1.0
S8 = 0.9171
A1 = 0.3385
A2 = 2.883
K1 = 16.0
K3 = -4.0
A1S3 = A1 * math.sqrt(3.0)


def _sc_body(za_h, dij_h, ii_h, jj_h, rows_h, rcov_h, st_h, out_h,
             nc_sh, e_sh, za_v, nc_v, rcov_v, st_v,
             iia, jja, da, damp_a, sema,
             iiB, jjB, dB, semI0, semI1, semI2,
             p0, rows0, semR0, p1, rows1, semR1,
             ebuf, isc, rd_t, c0_t, zero_v):
  cid = lax.axis_index("c")
  sid = lax.axis_index("s")
  wid = cid * NTILES + sid

  lane = lax.iota(jnp.int32, 16)
  zf = jnp.zeros((16,), jnp.float32)
  zi32 = jnp.zeros((16,), jnp.int32)

  pltpu.sync_copy(za_h, za_v)
  pltpu.sync_copy(rcov_h, rcov_v)
  pltpu.sync_copy(st_h, st_v)

  @pl.when(sid == 0)
  def _():
    def zi_(i, _):
      zero_v[pl.ds(i * 16, 16)] = zf
      return 0
    lax.fori_loop(0, ZCH // 16, zi_, 0)

    def zcopy(i, _):
      pltpu.sync_copy(zero_v, nc_sh.at[pl.ds(i * ZCH, ZCH)])
      pltpu.sync_copy(zero_v, e_sh.at[pl.ds(i * ZCH, ZCH)])
      return 0
    lax.fori_loop(0, N_ATOMS // ZCH, zcopy, 0)

  plsc.subcore_barrier()

  base_a = sid * EA

  def chunk_a(c, _):
    off = base_a + c * CA
    pltpu.async_copy(ii_h.at[pl.ds(off, CA)], iia, sema)
    pltpu.async_copy(jj_h.at[pl.ds(off, CA)], jja, sema)
    pltpu.async_copy(dij_h.at[pl.ds(off, CA)], da, sema)
    pltpu.make_async_copy(ii_h.at[pl.ds(off, CA)], iia, sema).wait()
    pltpu.make_async_copy(jj_h.at[pl.ds(off, CA)], jja, sema).wait()
    pltpu.make_async_copy(dij_h.at[pl.ds(off, CA)], da, sema).wait()

    def grp(g, _):
      o = g * 16
      ii = iia[pl.ds(o, 16)]
      jj = jja[pl.ds(o, 16)]
      d = da[pl.ds(o, 16)]
      zi = plsc.load_gather(za_v, [ii])
      zj = plsc.load_gather(za_v, [jj])
      rci = plsc.load_gather(rcov_v, [zi])
      rcj = plsc.load_gather(rcov_v, [zj])
      rr = (rci + rcj) * (BOHR) / d
      damp = 1.0 / (1.0 + jnp.exp(-K1 * (rr - 1.0)))
      damp_a[pl.ds(o, 16)] = damp
      return 0

    lax.fori_loop(0, CA // 16, grp, 0)
    pltpu.sync_copy(damp_a, nc_sh.at[iia], add=True)
    return 0

  with jax.named_scope("phaseA"):
    lax.fori_loop(0, EA // CA, chunk_a, 0)
  plsc.subcore_barrier()

  pltpu.sync_copy(nc_sh, nc_v)

  base_b = wid * EB
  isets = ((semI0,), (semI1,), (semI2,))
  rsets = ((p0, rows0, semR0), (p1, rows1, semR1))

  def idx_refs(s):
    return (iiB.at[pl.ds(s * CB, CB)], jjB.at[pl.ds(s * CB, CB)],
            dB.at[pl.ds(s * CB, CB)])

  def idx_fire(c, s):
    off = base_b + c * CB
    ii_r, jj_r, d_r = idx_refs(s)
    sem = isets[s][0]
    pltpu.async_copy(ii_h.at[pl.ds(off, CB)], ii_r, sem)
    pltpu.async_copy(jj_h.at[pl.ds(off, CB)], jj_r, sem)
    pltpu.async_copy(dij_h.at[pl.ds(off, CB)], d_r, sem)

  def idx_wait(c, s):
    off = base_b + c * CB
    ii_r, jj_r, d_r = idx_refs(s)
    sem = isets[s][0]
    pltpu.make_async_copy(ii_h.at[pl.ds(off, CB)], ii_r, sem).wait()
    pltpu.make_async_copy(jj_h.at[pl.ds(off, CB)], jj_r, sem).wait()
    pltpu.make_async_copy(dij_h.at[pl.ds(off, CB)], d_r, sem).wait()

  def rows_fire(s, r):
    PP, RR, semr = rsets[r]
    sbase = s * CB

    def pgrp(g, _):
      o = g * 16
      ii = iiB[pl.ds(sbase + o, 16)]
      jj = jjB[pl.ds(sbase + o, 16)]
      zi = plsc.load_gather(za_v, [ii])
      zj = plsc.load_gather(za_v, [jj])
      PP[pl.ds(o, 16)] = zi * NZ + zj
      return 0

    lax.fori_loop(0, CB // 16, pgrp, 0)
    pltpu.async_copy(rows_h.at[PP], RR, semr)

  def compute_chunk(s, r):
    PP, RR, semr = rsets[r]
    sbase = s * CB
    pltpu.make_async_copy(rows_h.at[PP], RR, semr).wait()

    def egrp(g, _):
      o = g * 16
      ii = iiB[pl.ds(sbase + o, 16)]
      jj = jjB[pl.ds(sbase + o, 16)]
      d = dB[pl.ds(sbase + o, 16)]
      zi = plsc.load_gather(za_v, [ii])
      zj = plsc.load_gather(za_v, [jj])
      nci = plsc.load_gather(nc_v, [ii])
      ncj = plsc.load_gather(nc_v, [jj])
      row = lane + o

      def ps1(k, m):
        col = 3 * k
        c0 = plsc.load_gather(RR, [row, zi32 + col])
        c1 = plsc.load_gather(RR, [row, zi32 + (col + 1)])
        c2 = plsc.load_gather(RR, [row, zi32 + (col + 2)])
        dd1 = c1 - nci
        dd2 = c2 - ncj
        rk = dd1 * dd1 + dd2 * dd2
        rd_t[pl.ds(k * 16, 16)] = rk
        c0_t[pl.ds(k * 16, 16)] = c0
        return jnp.minimum(m, rk)

      m = lax.fori_loop(0, NREF, ps1, jnp.full((16,), 1e30, jnp.float32))

      def ps2(k, carry):
        su, acc = carry
        rk = rd_t[pl.ds(k * 16, 16)]
        c0 = c0_t[pl.ds(k * 16, 16)]
        w = jnp.exp(K3 * (rk - m))
        return (su + w, acc + w * c0)

      su, acc = lax.fori_loop(0, NREF, ps2, (zf, zf))
      c6 = acc / su

      sti = plsc.load_gather(st_v, [zi])
      stj = plsc.load_gather(st_v, [zj])
      u = sti * stj
      c8 = (3.0 * c6) * (u * u)
      tmp = A1S3 * u + A2
      rr_ = d * (1.0 / BOHR)
      r2 = rr_ * rr_
      r6 = r2 * r2 * r2
      r8 = r6 * r2
      t2 = tmp * tmp
      t6 = t2 * t2 * t2
      t8 = t6 * t2
      e = (-0.5 * S6) * c6 / (r6 + t6) + (-0.5 * S8) * c8 / (r8 + t8)
      ebuf[pl.ds(o, 16)] = e
      isc[pl.ds(o, 16)] = ii
      return 0

    lax.fori_loop(0, CB // 16, egrp, 0)
    pltpu.sync_copy(ebuf, e_sh.at[isc], add=True)

  idx_fire(0, 0)
  idx_fire(1, 1)
  idx_wait(0, 0)
  rows_fire(0, 0)

  def outer(c6_, _):
    for b in range(6):
      c = c6_ * 6 + b

      @pl.when(c + 1 < NCH)
      def _():
        idx_wait(c + 1, (b + 1) % 3)
        rows_fire((b + 1) % 3, (b + 1) % 2)

      @pl.when(c < NCH)
      def _():
        compute_chunk(b % 3, b % 2)

      @pl.when(c + 2 < NCH)
      def _():
        idx_fire(c + 2, (b + 2) % 3)
    return 0

  with jax.named_scope("phaseB"):
    lax.fori_loop(0, (NCH + 5) // 6, outer, 0)
  plsc.subcore_barrier()

  @pl.when(sid == 0)
  def _():
    pltpu.sync_copy(e_sh, out_h.at[cid])


def _combine_body(p_ref, o_ref):
  o_ref[...] = p_ref[0, :] + p_ref[1, :]


def kernel(Za, Dij, idx_i, idx_j, c6ab, rcov, r2r4):
  Za = Za.astype(jnp.int32)
  idx_i = idx_i.astype(jnp.int32)
  idx_j = idx_j.astype(jnp.int32)
  Dij = Dij.astype(jnp.float32)
  rows = jnp.pad(c6ab.astype(jnp.float32).reshape(NZ * NZ, 75),
                 ((0, 0), (0, ROW - 75)))
  rcov_p = jnp.pad(rcov.astype(jnp.float32), (0, 1))
  st_p = jnp.pad(jnp.sqrt(r2r4.astype(jnp.float32)), (0, 1))

  mesh = plsc.VectorSubcoreMesh(core_axis_name="c", subcore_axis_name="s")
  sc = pl.kernel(
      _sc_body,
      out_type=jax.ShapeDtypeStruct((NCORES, N_ATOMS), jnp.float32),
      mesh=mesh,
      compiler_params=pltpu.CompilerParams(
          needs_layout_passes=False, use_tc_tiling_on_sc=False),
      scratch_types=[
          pltpu.VMEM_SHARED((N_ATOMS,), jnp.float32),
          pltpu.VMEM_SHARED((N_ATOMS,), jnp.float32),
          pltpu.VMEM((N_ATOMS,), jnp.int32),
          pltpu.VMEM((N_ATOMS,), jnp.float32),
          pltpu.VMEM((NZ + 1,), jnp.float32),
          pltpu.VMEM((NZ + 1,), jnp.float32),
          pltpu.VMEM((CA,), jnp.int32),
          pltpu.VMEM((CA,), jnp.int32),
          pltpu.VMEM((CA,), jnp.float32),
          pltpu.VMEM((CA,), jnp.float32),
          pltpu.SemaphoreType.DMA,
          pltpu.VMEM((3 * CB,), jnp.int32),
          pltpu.VMEM((3 * CB,), jnp.int32),
          pltpu.VMEM((3 * CB,), jnp.float32),
          pltpu.SemaphoreType.DMA,
          pltpu.SemaphoreType.DMA,
          pltpu.SemaphoreType.DMA,
          pltpu.VMEM((CB,), jnp.int32),
          pltpu.VMEM((CB, ROW), jnp.float32),
          pltpu.SemaphoreType.DMA,
          pltpu.VMEM((CB,), jnp.int32),
          pltpu.VMEM((CB, ROW), jnp.float32),
          pltpu.SemaphoreType.DMA,
          pltpu.VMEM((CB,), jnp.float32),
          pltpu.VMEM((CB,), jnp.int32),
          pltpu.VMEM((NREF * 16,), jnp.float32),
          pltpu.VMEM((NREF * 16,), jnp.float32),
          pltpu.VMEM((ZCH,), jnp.float32),
      ],
  )
  partials = sc(Za, Dij, idx_i, idx_j, rows, rcov_p, st_p)

  return pl.pallas_call(
      _combine_body,
      out_shape=jax.ShapeDtypeStruct((N_ATOMS,), jnp.float32),
  )(partials)


# speedup vs baseline: 1.0413x; 1.0413x over previous
import math

import jax
import jax.numpy as jnp
from jax import lax
from jax.experimental import pallas as pl
from jax.experimental.pallas import tpu as pltpu
from jax.experimental.pallas import tpu_sc as plsc

BOHR = 0.5291772108
N_ATOMS = 10000
N_EDGES = 640000
NZ = 95
NREF = 25
ROW = 80
NTILES = 16
NCORES = 2

EA = N_EDGES // NTILES
CA = 4000
EB = N_EDGES // (NCORES * NTILES)
CB = 400
NCH = EB // CB
ZCH = 2000

S6 = 1.0
S8 = 0.9171
A1 = 0.3385
A2 = 2.883
K1 = 16.0
K3 = -4.0
A1S3 = A1 * math.sqrt(3.0)


def _sc_body(za_h, dij_h, ii_h, jj_h, rows_h, rcov_h, st_h, out_h,
             nc_sh, e_sh, za_v, nc_v, rcov_v, st_v,
             iia, jja, da, damp_a, sema,
             iiB, jjB, dB, semI0, semI1,
             p0, rows0, semR0, p1, rows1, semR1,
             ebuf, isc, rd_t, c0_t, zero_v):
  cid = lax.axis_index("c")
  sid = lax.axis_index("s")
  wid = cid * NTILES + sid

  lane = lax.iota(jnp.int32, 16)
  zf = jnp.zeros((16,), jnp.float32)
  zi32 = jnp.zeros((16,), jnp.int32)

  pltpu.sync_copy(za_h, za_v)
  pltpu.sync_copy(rcov_h, rcov_v)
  pltpu.sync_copy(st_h, st_v)

  @pl.when(sid == 0)
  def _():
    def zi_(i, _):
      zero_v[pl.ds(i * 16, 16)] = zf
      return 0
    lax.fori_loop(0, ZCH // 16, zi_, 0)

    def zcopy(i, _):
      pltpu.sync_copy(zero_v, nc_sh.at[pl.ds(i * ZCH, ZCH)])
      pltpu.sync_copy(zero_v, e_sh.at[pl.ds(i * ZCH, ZCH)])
      return 0
    lax.fori_loop(0, N_ATOMS // ZCH, zcopy, 0)

  plsc.subcore_barrier()

  base_a = sid * EA

  def chunk_a(c, _):
    off = base_a + c * CA
    pltpu.async_copy(ii_h.at[pl.ds(off, CA)], iia, sema)
    pltpu.async_copy(jj_h.at[pl.ds(off, CA)], jja, sema)
    pltpu.async_copy(dij_h.at[pl.ds(off, CA)], da, sema)
    pltpu.make_async_copy(ii_h.at[pl.ds(off, CA)], iia, sema).wait()
    pltpu.make_async_copy(jj_h.at[pl.ds(off, CA)], jja, sema).wait()
    pltpu.make_async_copy(dij_h.at[pl.ds(off, CA)], da, sema).wait()

    def grp(g, _):
      for u in range(5):
        o = g * 80 + u * 16
        ii = iia[pl.ds(o, 16)]
        jj = jja[pl.ds(o, 16)]
        d = da[pl.ds(o, 16)]
        zi = plsc.load_gather(za_v, [ii])
        zj = plsc.load_gather(za_v, [jj])
        rci = plsc.load_gather(rcov_v, [zi])
        rcj = plsc.load_gather(rcov_v, [zj])
        rr = (rci + rcj) * (BOHR) / d
        damp = 1.0 / (1.0 + jnp.exp(-K1 * (rr - 1.0)))
        damp_a[pl.ds(o, 16)] = damp
      return 0

    lax.fori_loop(0, CA // 80, grp, 0)
    pltpu.sync_copy(damp_a, nc_sh.at[iia], add=True)
    return 0

  with jax.named_scope("phaseA"):
    lax.fori_loop(0, EA // CA, chunk_a, 0)
  plsc.subcore_barrier()

  pltpu.sync_copy(nc_sh, nc_v)

  base_b = wid * EB
  isets = ((semI0,), (semI1,))
  rsets = ((p0, rows0, semR0), (p1, rows1, semR1))

  def idx_refs(s):
    return (iiB.at[pl.ds(s * CB, CB)], jjB.at[pl.ds(s * CB, CB)],
            dB.at[pl.ds(s * CB, CB)])

  def idx_fire(c, s):
    off = base_b + c * CB
    ii_r, jj_r, d_r = idx_refs(s)
    sem = isets[s][0]
    pltpu.async_copy(ii_h.at[pl.ds(off, CB)], ii_r, sem)
    pltpu.async_copy(jj_h.at[pl.ds(off, CB)], jj_r, sem)
    pltpu.async_copy(dij_h.at[pl.ds(off, CB)], d_r, sem)

  def idx_wait(c, s):
    off = base_b + c * CB
    ii_r, jj_r, d_r = idx_refs(s)
    sem = isets[s][0]
    pltpu.make_async_copy(ii_h.at[pl.ds(off, CB)], ii_r, sem).wait()
    pltpu.make_async_copy(jj_h.at[pl.ds(off, CB)], jj_r, sem).wait()
    pltpu.make_async_copy(dij_h.at[pl.ds(off, CB)], d_r, sem).wait()

  def rows_fire(s, r):
    PP, RR, semr = rsets[r]
    sbase = s * CB

    def pgrp(g, _):
      o = g * 16
      ii = iiB[pl.ds(sbase + o, 16)]
      jj = jjB[pl.ds(sbase + o, 16)]
      zi = plsc.load_gather(za_v, [ii])
      zj = plsc.load_gather(za_v, [jj])
      PP[pl.ds(o, 16)] = zi * NZ + zj
      return 0

    lax.fori_loop(0, CB // 16, pgrp, 0)
    pltpu.async_copy(rows_h.at[PP], RR, semr)

  def compute_chunk(s, r):
    PP, RR, semr = rsets[r]
    sbase = s * CB
    pltpu.make_async_copy(rows_h.at[PP], RR, semr).wait()

    def egrp(g, _):
      o = g * 16
      ii = iiB[pl.ds(sbase + o, 16)]
      jj = jjB[pl.ds(sbase + o, 16)]
      d = dB[pl.ds(sbase + o, 16)]
      zi = plsc.load_gather(za_v, [ii])
      zj = plsc.load_gather(za_v, [jj])
      nci = plsc.load_gather(nc_v, [ii])
      ncj = plsc.load_gather(nc_v, [jj])
      row = lane + o

      m = jnp.full((16,), 1e30, jnp.float32)
      for k in range(NREF):
        col = 3 * k
        c0 = plsc.load_gather(RR, [row, zi32 + col])
        c1 = plsc.load_gather(RR, [row, zi32 + (col + 1)])
        c2 = plsc.load_gather(RR, [row, zi32 + (col + 2)])
        dd1 = c1 - nci
        dd2 = c2 - ncj
        rk = dd1 * dd1 + dd2 * dd2
        rd_t[pl.ds(k * 16, 16)] = rk
        c0_t[pl.ds(k * 16, 16)] = c0
        m = jnp.minimum(m, rk)

      su = zf
      acc = zf
      for k in range(NREF):
        rk = rd_t[pl.ds(k * 16, 16)]
        c0 = c0_t[pl.ds(k * 16, 16)]
        w = jnp.exp(K3 * (rk - m))
        su = su + w
        acc = acc + w * c0
      c6 = acc / su

      sti = plsc.load_gather(st_v, [zi])
      stj = plsc.load_gather(st_v, [zj])
      u = sti * stj
      c8 = (3.0 * c6) * (u * u)
      tmp = A1S3 * u + A2
      rr_ = d * (1.0 / BOHR)
      r2 = rr_ * rr_
      r6 = r2 * r2 * r2
      r8 = r6 * r2
      t2 = tmp * tmp
      t6 = t2 * t2 * t2
      t8 = t6 * t2
      e = (-0.5 * S6) * c6 / (r6 + t6) + (-0.5 * S8) * c8 / (r8 + t8)
      ebuf[pl.ds(o, 16)] = e
      isc[pl.ds(o, 16)] = ii
      return 0

    lax.fori_loop(0, CB // 16, egrp, 0)
    pltpu.sync_copy(ebuf, e_sh.at[isc], add=True)

  idx_fire(0, 0)
  idx_fire(1, 1)
  idx_wait(0, 0)
  rows_fire(0, 0)

  def outer(c2_, _):
    for b in range(2):
      c = c2_ * 2 + b

      @pl.when(c + 1 < NCH)
      def _():
        idx_wait(c + 1, (b + 1) % 2)
        rows_fire((b + 1) % 2, (b + 1) % 2)

      compute_chunk(b, b)

      @pl.when(c + 2 < NCH)
      def _():
        idx_fire(c + 2, b)
    return 0

  with jax.named_scope("phaseB"):
    lax.fori_loop(0, NCH // 2, outer, 0)
  plsc.subcore_barrier()

  @pl.when(sid == 0)
  def _():
    pltpu.sync_copy(e_sh, out_h.at[cid])


def _combine_body(p_ref, o_ref):
  o_ref[...] = p_ref[0, :] + p_ref[1, :]


def kernel(Za, Dij, idx_i, idx_j, c6ab, rcov, r2r4):
  Za = Za.astype(jnp.int32)
  idx_i = idx_i.astype(jnp.int32)
  idx_j = idx_j.astype(jnp.int32)
  Dij = Dij.astype(jnp.float32)
  rows = jnp.pad(c6ab.astype(jnp.float32).reshape(NZ * NZ, 75),
                 ((0, 0), (0, ROW - 75)))
  rcov_p = jnp.pad(rcov.astype(jnp.float32), (0, 1))
  st_p = jnp.pad(jnp.sqrt(r2r4.astype(jnp.float32)), (0, 1))

  mesh = plsc.VectorSubcoreMesh(core_axis_name="c", subcore_axis_name="s")
  sc = pl.kernel(
      _sc_body,
      out_type=jax.ShapeDtypeStruct((NCORES, N_ATOMS), jnp.float32),
      mesh=mesh,
      compiler_params=pltpu.CompilerParams(
          needs_layout_passes=False, use_tc_tiling_on_sc=False),
      scratch_types=[
          pltpu.VMEM_SHARED((N_ATOMS,), jnp.float32),
          pltpu.VMEM_SHARED((N_ATOMS,), jnp.float32),
          pltpu.VMEM((N_ATOMS,), jnp.int32),
          pltpu.VMEM((N_ATOMS,), jnp.float32),
          pltpu.VMEM((NZ + 1,), jnp.float32),
          pltpu.VMEM((NZ + 1,), jnp.float32),
          pltpu.VMEM((CA,), jnp.int32),
          pltpu.VMEM((CA,), jnp.int32),
          pltpu.VMEM((CA,), jnp.float32),
          pltpu.VMEM((CA,), jnp.float32),
          pltpu.SemaphoreType.DMA,
          pltpu.VMEM((2 * CB,), jnp.int32),
          pltpu.VMEM((2 * CB,), jnp.int32),
          pltpu.VMEM((2 * CB,), jnp.float32),
          pltpu.SemaphoreType.DMA,
          pltpu.SemaphoreType.DMA,
          pltpu.VMEM((CB,), jnp.int32),
          pltpu.VMEM((CB, ROW), jnp.float32),
          pltpu.SemaphoreType.DMA,
          pltpu.VMEM((CB,), jnp.int32),
          pltpu.VMEM((CB, ROW), jnp.float32),
          pltpu.SemaphoreType.DMA,
          pltpu.VMEM((CB,), jnp.float32),
          pltpu.VMEM((CB,), jnp.int32),
          pltpu.VMEM((NREF * 16,), jnp.float32),
          pltpu.VMEM((NREF * 16,), jnp.float32),
          pltpu.VMEM((ZCH,), jnp.float32),
      ],
  )
  partials = sc(Za, Dij, idx_i, idx_j, rows, rcov_p, st_p)

  return pl.pallas_call(
      _combine_body,
      out_shape=jax.ShapeDtypeStruct((N_ATOMS,), jnp.float32),
  )(partials)


# speedup vs baseline: 1.3835x; 1.3286x over previous
import math

import jax
import jax.numpy as jnp
from jax import lax
from jax.experimental import pallas as pl
from jax.experimental.pallas import tpu as pltpu
from jax.experimental.pallas import tpu_sc as plsc

BOHR = 0.5291772108
N_ATOMS = 10000
N_EDGES = 640000
NZ = 95
NREF = 25
ROW = 80
NTILES = 16
NCORES = 2

EA = N_EDGES // NTILES
CA = 4000
EB = N_EDGES // (NCORES * NTILES)
CB = 400
NCH = EB // CB
ZCH = 2000

S6 = 1.0
S8 = 0.9171
A1 = 0.3385
A2 = 2.883
K1 = 16.0
K3 = -4.0
A1S3 = A1 * math.sqrt(3.0)


def _sc_body(za_h, dij_h, ii_h, jj_h, rows_h, rcov_h, st_h, out_h,
             nc_sh, e_sh, za_v, nc_v, rcov_v, st_v,
             iia, jja, da, damp_a, sema,
             iiB, jjB, dB, semI0, semI1,
             p0, rows0, semR0, p1, rows1, semR1,
             ebuf, isc, rd_t, c0_t, zero_v):
  cid = lax.axis_index("c")
  sid = lax.axis_index("s")
  wid = cid * NTILES + sid

  lane = lax.iota(jnp.int32, 16)
  zf = jnp.zeros((16,), jnp.float32)
  zi32 = jnp.zeros((16,), jnp.int32)

  pltpu.sync_copy(za_h, za_v)
  pltpu.sync_copy(rcov_h, rcov_v)
  pltpu.sync_copy(st_h, st_v)

  @pl.when(sid == 0)
  def _():
    def zi_(i, _):
      zero_v[pl.ds(i * 16, 16)] = zf
      return 0
    lax.fori_loop(0, ZCH // 16, zi_, 0)

    def zcopy(i, _):
      pltpu.sync_copy(zero_v, nc_sh.at[pl.ds(i * ZCH, ZCH)])
      pltpu.sync_copy(zero_v, e_sh.at[pl.ds(i * ZCH, ZCH)])
      return 0
    lax.fori_loop(0, N_ATOMS // ZCH, zcopy, 0)

  plsc.subcore_barrier()

  base_a = sid * EA

  def chunk_a(c, _):
    off = base_a + c * CA
    pltpu.async_copy(ii_h.at[pl.ds(off, CA)], iia, sema)
    pltpu.async_copy(jj_h.at[pl.ds(off, CA)], jja, sema)
    pltpu.async_copy(dij_h.at[pl.ds(off, CA)], da, sema)
    pltpu.make_async_copy(ii_h.at[pl.ds(off, CA)], iia, sema).wait()
    pltpu.make_async_copy(jj_h.at[pl.ds(off, CA)], jja, sema).wait()
    pltpu.make_async_copy(dij_h.at[pl.ds(off, CA)], da, sema).wait()

    def grp(g, _):
      for u in range(5):
        o = g * 80 + u * 16
        ii = iia[pl.ds(o, 16)]
        jj = jja[pl.ds(o, 16)]
        d = da[pl.ds(o, 16)]
        zi = plsc.load_gather(za_v, [ii])
        zj = plsc.load_gather(za_v, [jj])
        rci = plsc.load_gather(rcov_v, [zi])
        rcj = plsc.load_gather(rcov_v, [zj])
        rr = (rci + rcj) * (BOHR) / d
        damp = 1.0 / (1.0 + jnp.exp(-K1 * (rr - 1.0)))
        damp_a[pl.ds(o, 16)] = damp
      return 0

    lax.fori_loop(0, CA // 80, grp, 0)
    pltpu.sync_copy(damp_a, nc_sh.at[iia], add=True)
    return 0

  with jax.named_scope("phaseA"):
    lax.fori_loop(0, EA // CA, chunk_a, 0)
  plsc.subcore_barrier()

  pltpu.sync_copy(nc_sh, nc_v)

  base_b = wid * EB
  isets = ((semI0,), (semI1,))
  rsets = ((p0, rows0, semR0), (p1, rows1, semR1))

  def idx_refs(s):
    return (iiB.at[pl.ds(s * CB, CB)], jjB.at[pl.ds(s * CB, CB)],
            dB.at[pl.ds(s * CB, CB)])

  def idx_fire(c, s):
    off = base_b + c * CB
    ii_r, jj_r, d_r = idx_refs(s)
    sem = isets[s][0]
    pltpu.async_copy(ii_h.at[pl.ds(off, CB)], ii_r, sem)
    pltpu.async_copy(jj_h.at[pl.ds(off, CB)], jj_r, sem)
    pltpu.async_copy(dij_h.at[pl.ds(off, CB)], d_r, sem)

  def idx_wait(c, s):
    off = base_b + c * CB
    ii_r, jj_r, d_r = idx_refs(s)
    sem = isets[s][0]
    pltpu.make_async_copy(ii_h.at[pl.ds(off, CB)], ii_r, sem).wait()
    pltpu.make_async_copy(jj_h.at[pl.ds(off, CB)], jj_r, sem).wait()
    pltpu.make_async_copy(dij_h.at[pl.ds(off, CB)], d_r, sem).wait()

  def rows_fire(s, r):
    PP, RR, semr = rsets[r]
    sbase = s * CB

    def pgrp(g, _):
      o = g * 16
      ii = iiB[pl.ds(sbase + o, 16)]
      jj = jjB[pl.ds(sbase + o, 16)]
      zi = plsc.load_gather(za_v, [ii])
      zj = plsc.load_gather(za_v, [jj])
      PP[pl.ds(o, 16)] = zi * NZ + zj
      return 0

    lax.fori_loop(0, CB // 16, pgrp, 0)
    pltpu.async_copy(rows_h.at[PP], RR, semr)

  def compute_chunk(s, r):
    PP, RR, semr = rsets[r]
    sbase = s * CB
    pltpu.make_async_copy(rows_h.at[PP], RR, semr).wait()

    def egrp(g, _):
      o = g * 16
      ii = iiB[pl.ds(sbase + o, 16)]
      jj = jjB[pl.ds(sbase + o, 16)]
      d = dB[pl.ds(sbase + o, 16)]
      zi = plsc.load_gather(za_v, [ii])
      zj = plsc.load_gather(za_v, [jj])
      nci = plsc.load_gather(nc_v, [ii])
      ncj = plsc.load_gather(nc_v, [jj])
      row = lane + o

      m = jnp.full((16,), 1e30, jnp.float32)
      for k in range(NREF):
        t = lane + k
        vrot = jnp.where(t >= NREF, t - NREF, t)
        colv = 3 * vrot
        slotv = vrot * 16 + lane
        c0 = plsc.load_gather(RR, [row, colv])
        c1 = plsc.load_gather(RR, [row, colv + 1])
        c2 = plsc.load_gather(RR, [row, colv + 2])
        dd1 = c1 - nci
        dd2 = c2 - ncj
        rk = dd1 * dd1 + dd2 * dd2
        plsc.store_scatter(rd_t, [slotv], rk)
        plsc.store_scatter(c0_t, [slotv], c0)
        m = jnp.minimum(m, rk)

      su = zf
      acc = zf
      for k in range(NREF):
        rk = rd_t[pl.ds(k * 16, 16)]
        c0 = c0_t[pl.ds(k * 16, 16)]
        w = jnp.exp(K3 * (rk - m))
        su = su + w
        acc = acc + w * c0
      c6 = acc / su

      sti = plsc.load_gather(st_v, [zi])
      stj = plsc.load_gather(st_v, [zj])
      u = sti * stj
      c8 = (3.0 * c6) * (u * u)
      tmp = A1S3 * u + A2
      rr_ = d * (1.0 / BOHR)
      r2 = rr_ * rr_
      r6 = r2 * r2 * r2
      r8 = r6 * r2
      t2 = tmp * tmp
      t6 = t2 * t2 * t2
      t8 = t6 * t2
      e = (-0.5 * S6) * c6 / (r6 + t6) + (-0.5 * S8) * c8 / (r8 + t8)
      ebuf[pl.ds(o, 16)] = e
      isc[pl.ds(o, 16)] = ii
      return 0

    lax.fori_loop(0, CB // 16, egrp, 0)
    pltpu.sync_copy(ebuf, e_sh.at[isc], add=True)

  idx_fire(0, 0)
  idx_fire(1, 1)
  idx_wait(0, 0)
  rows_fire(0, 0)

  def outer(c2_, _):
    for b in range(2):
      c = c2_ * 2 + b

      @pl.when(c + 1 < NCH)
      def _():
        idx_wait(c + 1, (b + 1) % 2)
        rows_fire((b + 1) % 2, (b + 1) % 2)

      compute_chunk(b, b)

      @pl.when(c + 2 < NCH)
      def _():
        idx_fire(c + 2, b)
    return 0

  with jax.named_scope("phaseB"):
    lax.fori_loop(0, NCH // 2, outer, 0)
  plsc.subcore_barrier()

  @pl.when(sid == 0)
  def _():
    pltpu.sync_copy(e_sh, out_h.at[cid])


def _combine_body(p_ref, o_ref):
  o_ref[...] = p_ref[0, :] + p_ref[1, :]


def kernel(Za, Dij, idx_i, idx_j, c6ab, rcov, r2r4):
  Za = Za.astype(jnp.int32)
  idx_i = idx_i.astype(jnp.int32)
  idx_j = idx_j.astype(jnp.int32)
  Dij = Dij.astype(jnp.float32)
  rows = jnp.pad(c6ab.astype(jnp.float32).reshape(NZ * NZ, 75),
                 ((0, 0), (0, ROW - 75)))
  rcov_p = jnp.pad(rcov.astype(jnp.float32), (0, 1))
  st_p = jnp.pad(jnp.sqrt(r2r4.astype(jnp.float32)), (0, 1))

  mesh = plsc.VectorSubcoreMesh(core_axis_name="c", subcore_axis_name="s")
  sc = pl.kernel(
      _sc_body,
      out_type=jax.ShapeDtypeStruct((NCORES, N_ATOMS), jnp.float32),
      mesh=mesh,
      compiler_params=pltpu.CompilerParams(
          needs_layout_passes=False, use_tc_tiling_on_sc=False),
      scratch_types=[
          pltpu.VMEM_SHARED((N_ATOMS,), jnp.float32),
          pltpu.VMEM_SHARED((N_ATOMS,), jnp.float32),
          pltpu.VMEM((N_ATOMS,), jnp.int32),
          pltpu.VMEM((N_ATOMS,), jnp.float32),
          pltpu.VMEM((NZ + 1,), jnp.float32),
          pltpu.VMEM((NZ + 1,), jnp.float32),
          pltpu.VMEM((CA,), jnp.int32),
          pltpu.VMEM((CA,), jnp.int32),
          pltpu.VMEM((CA,), jnp.float32),
          pltpu.VMEM((CA,), jnp.float32),
          pltpu.SemaphoreType.DMA,
          pltpu.VMEM((2 * CB,), jnp.int32),
          pltpu.VMEM((2 * CB,), jnp.int32),
          pltpu.VMEM((2 * CB,), jnp.float32),
          pltpu.SemaphoreType.DMA,
          pltpu.SemaphoreType.DMA,
          pltpu.VMEM((CB,), jnp.int32),
          pltpu.VMEM((CB, ROW), jnp.float32),
          pltpu.SemaphoreType.DMA,
          pltpu.VMEM((CB,), jnp.int32),
          pltpu.VMEM((CB, ROW), jnp.float32),
          pltpu.SemaphoreType.DMA,
          pltpu.VMEM((CB,), jnp.float32),
          pltpu.VMEM((CB,), jnp.int32),
          pltpu.VMEM((NREF * 16,), jnp.float32),
          pltpu.VMEM((NREF * 16,), jnp.float32),
          pltpu.VMEM((ZCH,), jnp.float32),
      ],
  )
  partials = sc(Za, Dij, idx_i, idx_j, rows, rcov_p, st_p)

  return pl.pallas_call(
      _combine_body,
      out_shape=jax.ShapeDtypeStruct((N_ATOMS,), jnp.float32),
  )(partials)


# speedup vs baseline: 1.3842x; 1.0005x over previous
import math

import jax
import jax.numpy as jnp
from jax import lax
from jax.experimental import pallas as pl
from jax.experimental.pallas import tpu as pltpu
from jax.experimental.pallas import tpu_sc as plsc

BOHR = 0.5291772108
N_ATOMS = 10000
N_EDGES = 640000
NZ = 95
NREF = 25
ROW = 80
NTILES = 16
NCORES = 2

EA = N_EDGES // NTILES
CA = 4000
EB = N_EDGES // (NCORES * NTILES)
CB = 400
NCH = EB // CB
ZCH = 2000

S6 = 1.0
S8 = 0.9171
A1 = 0.3385
A2 = 2.883
K1 = 16.0
K3 = -4.0
A1S3 = A1 * math.sqrt(3.0)


def _sc_body(za_h, dij_h, ii_h, jj_h, rows_h, rcov_h, st_h, out_h,
             nc_sh, e_sh, za_v, nc_v, rcov_v, st_v,
             iia, jja, da, damp_a, sema,
             iiB, jjB, dB, semI0, semI1,
             p0, rows0, semR0, p1, rows1, semR1,
             ebuf, isc, rd_t, c0_t, zero_v):
  cid = lax.axis_index("c")
  sid = lax.axis_index("s")
  wid = cid * NTILES + sid

  lane = lax.iota(jnp.int32, 16)
  zf = jnp.zeros((16,), jnp.float32)

  pltpu.sync_copy(za_h, za_v)
  pltpu.sync_copy(rcov_h, rcov_v)
  pltpu.sync_copy(st_h, st_v)

  @pl.when(sid == 0)
  def _():
    def zi_(i, _):
      zero_v[pl.ds(i * 16, 16)] = zf
      return 0
    lax.fori_loop(0, ZCH // 16, zi_, 0)

    def zcopy(i, _):
      pltpu.sync_copy(zero_v, nc_sh.at[pl.ds(i * ZCH, ZCH)])
      pltpu.sync_copy(zero_v, e_sh.at[pl.ds(i * ZCH, ZCH)])
      return 0
    lax.fori_loop(0, N_ATOMS // ZCH, zcopy, 0)

  plsc.subcore_barrier()

  base_a = sid * EA

  def chunk_a(c, _):
    off = base_a + c * CA
    pltpu.async_copy(ii_h.at[pl.ds(off, CA)], iia, sema)
    pltpu.async_copy(jj_h.at[pl.ds(off, CA)], jja, sema)
    pltpu.async_copy(dij_h.at[pl.ds(off, CA)], da, sema)
    pltpu.make_async_copy(ii_h.at[pl.ds(off, CA)], iia, sema).wait()
    pltpu.make_async_copy(jj_h.at[pl.ds(off, CA)], jja, sema).wait()
    pltpu.make_async_copy(dij_h.at[pl.ds(off, CA)], da, sema).wait()

    def grp(g, _):
      for u in range(5):
        o = g * 80 + u * 16
        ii = iia[pl.ds(o, 16)]
        jj = jja[pl.ds(o, 16)]
        d = da[pl.ds(o, 16)]
        zi = plsc.load_gather(za_v, [ii])
        zj = plsc.load_gather(za_v, [jj])
        rci = plsc.load_gather(rcov_v, [zi])
        rcj = plsc.load_gather(rcov_v, [zj])
        rr = (rci + rcj) * (BOHR) / d
        damp = 1.0 / (1.0 + jnp.exp(-K1 * (rr - 1.0)))
        damp_a[pl.ds(o, 16)] = damp
      return 0

    lax.fori_loop(0, CA // 80, grp, 0)
    pltpu.sync_copy(damp_a, nc_sh.at[iia], add=True)
    return 0

  with jax.named_scope("phaseA"):
    lax.fori_loop(0, EA // CA, chunk_a, 0)
  plsc.subcore_barrier()

  pltpu.sync_copy(nc_sh, nc_v)

  base_b = wid * EB
  isets = ((semI0,), (semI1,))
  rsets = ((p0, rows0, semR0), (p1, rows1, semR1))

  def idx_refs(s):
    return (iiB.at[pl.ds(s * CB, CB)], jjB.at[pl.ds(s * CB, CB)],
            dB.at[pl.ds(s * CB, CB)])

  def idx_fire(c, s):
    off = base_b + c * CB
    ii_r, jj_r, d_r = idx_refs(s)
    sem = isets[s][0]
    pltpu.async_copy(ii_h.at[pl.ds(off, CB)], ii_r, sem)
    pltpu.async_copy(jj_h.at[pl.ds(off, CB)], jj_r, sem)
    pltpu.async_copy(dij_h.at[pl.ds(off, CB)], d_r, sem)

  def idx_wait(c, s):
    off = base_b + c * CB
    ii_r, jj_r, d_r = idx_refs(s)
    sem = isets[s][0]
    pltpu.make_async_copy(ii_h.at[pl.ds(off, CB)], ii_r, sem).wait()
    pltpu.make_async_copy(jj_h.at[pl.ds(off, CB)], jj_r, sem).wait()
    pltpu.make_async_copy(dij_h.at[pl.ds(off, CB)], d_r, sem).wait()

  def rows_fire(s, r):
    PP, RR, semr = rsets[r]
    sbase = s * CB

    def pgrp(g, _):
      o = g * 16
      ii = iiB[pl.ds(sbase + o, 16)]
      jj = jjB[pl.ds(sbase + o, 16)]
      zi = plsc.load_gather(za_v, [ii])
      zj = plsc.load_gather(za_v, [jj])
      PP[pl.ds(o, 16)] = zi * NZ + zj
      return 0

    lax.fori_loop(0, CB // 16, pgrp, 0)
    pltpu.async_copy(rows_h.at[PP], RR, semr)

  def compute_chunk(s, r):
    PP, RR, semr = rsets[r]
    sbase = s * CB
    pltpu.make_async_copy(rows_h.at[PP], RR, semr).wait()

    def egrp(g, _):
      o = g * 16
      ii = iiB[pl.ds(sbase + o, 16)]
      jj = jjB[pl.ds(sbase + o, 16)]
      d = dB[pl.ds(sbase + o, 16)]
      zi = plsc.load_gather(za_v, [ii])
      zj = plsc.load_gather(za_v, [jj])
      nci = plsc.load_gather(nc_v, [ii])
      ncj = plsc.load_gather(nc_v, [jj])
      row = lane + o

      m = jnp.full((16,), 1e30, jnp.float32)
      for k in range(NREF):
        t = lane + k
        vrot = jnp.where(t >= NREF, t - NREF, t)
        colv = 3 * vrot
        slotv = vrot * 16 + lane
        c0 = plsc.load_gather(RR, [row, colv])
        c1 = plsc.load_gather(RR, [row, colv + 1])
        c2 = plsc.load_gather(RR, [row, colv + 2])
        dd1 = c1 - nci
        dd2 = c2 - ncj
        rk = dd1 * dd1 + dd2 * dd2
        plsc.store_scatter(rd_t, [slotv], rk)
        plsc.store_scatter(c0_t, [slotv], c0)
        m = jnp.minimum(m, rk)

      su = zf
      acc = zf
      for k in range(NREF):
        rk = rd_t[pl.ds(k * 16, 16)]
        c0 = c0_t[pl.ds(k * 16, 16)]
        w = jnp.exp(K3 * (rk - m))
        su = su + w
        acc = acc + w * c0
      c6 = acc / su

      sti = plsc.load_gather(st_v, [zi])
      stj = plsc.load_gather(st_v, [zj])
      u = sti * stj
      c8 = (3.0 * c6) * (u * u)
      tmp = A1S3 * u + A2
      rr_ = d * (1.0 / BOHR)
      r2 = rr_ * rr_
      r6 = r2 * r2 * r2
      r8 = r6 * r2
      t2 = tmp * tmp
      t6 = t2 * t2 * t2
      t8 = t6 * t2
      e = (-0.5 * S6) * c6 / (r6 + t6) + (-0.5 * S8) * c8 / (r8 + t8)
      ebuf[pl.ds(o, 16)] = e
      isc[pl.ds(o, 16)] = ii
      return 0

    lax.fori_loop(0, CB // 16, egrp, 0)
    pltpu.sync_copy(ebuf, e_sh.at[isc], add=True)

  idx_fire(0, 0)
  idx_fire(1, 1)
  idx_wait(0, 0)
  rows_fire(0, 0)

  def outer(c2_, _):
    for b in range(2):
      c = c2_ * 2 + b

      @pl.when(c + 1 < NCH)
      def _():
        idx_wait(c + 1, (b + 1) % 2)
        rows_fire((b + 1) % 2, (b + 1) % 2)

      compute_chunk(b, b)

      @pl.when(c + 2 < NCH)
      def _():
        idx_fire(c + 2, b)
    return 0

  with jax.named_scope("phaseB"):
    lax.fori_loop(0, NCH // 2, outer, 0)
  plsc.subcore_barrier()

  @pl.when(sid == 0)
  def _():
    pltpu.sync_copy(e_sh, out_h.at[cid])


def _combine_body(p_ref, o_ref):
  o_ref[...] = p_ref[0, :] + p_ref[1, :]


def kernel(Za, Dij, idx_i, idx_j, c6ab, rcov, r2r4):
  Za = Za.astype(jnp.int32)
  idx_i = idx_i.astype(jnp.int32)
  idx_j = idx_j.astype(jnp.int32)
  Dij = Dij.astype(jnp.float32)
  rows = jnp.pad(c6ab.astype(jnp.float32).reshape(NZ * NZ, 75),
                 ((0, 0), (0, ROW - 75)))
  rcov_p = jnp.pad(rcov.astype(jnp.float32), (0, 1))
  st_p = jnp.pad(jnp.sqrt(r2r4.astype(jnp.float32)), (0, 1))

  mesh = plsc.VectorSubcoreMesh(core_axis_name="c", subcore_axis_name="s")
  sc = pl.kernel(
      _sc_body,
      out_type=jax.ShapeDtypeStruct((NCORES, N_ATOMS), jnp.float32),
      mesh=mesh,
      compiler_params=pltpu.CompilerParams(
          needs_layout_passes=False, use_tc_tiling_on_sc=False),
      scratch_types=[
          pltpu.VMEM_SHARED((N_ATOMS,), jnp.float32),
          pltpu.VMEM_SHARED((N_ATOMS,), jnp.float32),
          pltpu.VMEM((N_ATOMS,), jnp.int32),
          pltpu.VMEM((N_ATOMS,), jnp.float32),
          pltpu.VMEM((NZ + 1,), jnp.float32),
          pltpu.VMEM((NZ + 1,), jnp.float32),
          pltpu.VMEM((CA,), jnp.int32),
          pltpu.VMEM((CA,), jnp.int32),
          pltpu.VMEM((CA,), jnp.float32),
          pltpu.VMEM((CA,), jnp.float32),
          pltpu.SemaphoreType.DMA,
          pltpu.VMEM((2 * CB,), jnp.int32),
          pltpu.VMEM((2 * CB,), jnp.int32),
          pltpu.VMEM((2 * CB,), jnp.float32),
          pltpu.SemaphoreType.DMA,
          pltpu.SemaphoreType.DMA,
          pltpu.VMEM((CB,), jnp.int32),
          pltpu.VMEM((CB, ROW), jnp.float32),
          pltpu.SemaphoreType.DMA,
          pltpu.VMEM((CB,), jnp.int32),
          pltpu.VMEM((CB, ROW), jnp.float32),
          pltpu.SemaphoreType.DMA,
          pltpu.VMEM((CB,), jnp.float32),
          pltpu.VMEM((CB,), jnp.int32),
          pltpu.VMEM((NREF * 16,), jnp.float32),
          pltpu.VMEM((NREF * 16,), jnp.float32),
          pltpu.VMEM((ZCH,), jnp.float32),
      ],
  )
  partials = sc(Za, Dij, idx_i, idx_j, rows, rcov_p, st_p)

  return pl.pallas_call(
      _combine_body,
      out_shape=jax.ShapeDtypeStruct((N_ATOMS,), jnp.float32),
  )(partials)
